# baseline (speedup 1.0000x reference)
import functools

import jax
import jax.numpy as jnp
from jax import lax
from jax.experimental import pallas as pl
from jax.experimental.pallas import tpu as pltpu
from jax.experimental.pallas import tpu_sc as plsc

B, S, D = 4, 4096, 1024
NC, NS = 2, 16
NW = NC * NS
SEQ_PER_W = S // NW
CHUNK = 32
N_CHUNKS = SEQ_PER_W // CHUNK
CW = CHUNK * D
LANES = 16
UNROLL = 8

_mesh = plsc.VectorSubcoreMesh(core_axis_name="c", subcore_axis_name="s")


@functools.partial(
    pl.kernel,
    out_type=jax.ShapeDtypeStruct((B * S * D,), jnp.float32),
    mesh=_mesh,
    scratch_types=[
        pltpu.VMEM((CW,), jnp.float32),
        pltpu.VMEM((CW,), jnp.float32),
    ],
)
def _pe_add(te_hbm, enc_hbm, out_hbm, enc_buf, te_buf):
    wid = lax.axis_index("s") * NC + lax.axis_index("c")
    s_base = wid * SEQ_PER_W
    for c in range(N_CHUNKS):
        e0 = (s_base + c * CHUNK) * D
        pltpu.sync_copy(enc_hbm.at[pl.ds(e0, CW)], enc_buf)
        for b in range(B):
            r0 = b * (S * D) + e0
            pltpu.sync_copy(te_hbm.at[pl.ds(r0, CW)], te_buf)

            def body(i, carry):
                base = i * (LANES * UNROLL)
                for u in range(UNROLL):
                    off = base + u * LANES
                    te_buf[pl.ds(off, LANES)] = (
                        te_buf[pl.ds(off, LANES)] + enc_buf[pl.ds(off, LANES)]
                    )
                return carry

            lax.fori_loop(0, CW // (LANES * UNROLL), body, 0)
            pltpu.sync_copy(te_buf, out_hbm.at[pl.ds(r0, CW)])


def kernel(token_embeddings, encoding):
    out = _pe_add(token_embeddings.reshape(-1), encoding.reshape(-1))
    return out.reshape(B, S, D)


# speedup vs baseline: 1.1901x; 1.1901x over previous
import functools

import jax
import jax.numpy as jnp
from jax import lax
from jax.experimental import pallas as pl
from jax.experimental.pallas import tpu as pltpu
from jax.experimental.pallas import tpu_sc as plsc

B, S, D = 4, 4096, 1024
NC, NS = 2, 16
NW = NC * NS
SEQ_PER_W = S // NW
CHUNK = 8
N_CHUNKS = SEQ_PER_W // CHUNK
CW = CHUNK * D
LANES = 16
GROUP = 8
NSLOT = 3

_mesh = plsc.VectorSubcoreMesh(core_axis_name="c", subcore_axis_name="s")


@functools.partial(
    pl.kernel,
    out_type=jax.ShapeDtypeStruct((B * S * D,), jnp.float32),
    mesh=_mesh,
    scratch_types=[
        [pltpu.VMEM((B * CW,), jnp.float32) for _ in range(NSLOT)],
        [pltpu.VMEM((CW,), jnp.float32) for _ in range(2)],
        [pltpu.SemaphoreType.DMA for _ in range(NSLOT)],
        [pltpu.SemaphoreType.DMA for _ in range(NSLOT)],
        [pltpu.SemaphoreType.DMA for _ in range(2)],
    ],
)
def _pe_add(te_hbm, enc_hbm, out_hbm, slots, ebufs, in_sems, out_sems, e_sems):
    wid = lax.axis_index("s") * NC + lax.axis_index("c")
    e_base = wid * (SEQ_PER_W * D)

    def issue_in(c):
        slot = c % NSLOT
        e0 = e_base + c * CW
        return [
            pltpu.async_copy(
                te_hbm.at[pl.ds(b * (S * D) + e0, CW)],
                slots[slot].at[pl.ds(b * CW, CW)],
                in_sems[slot],
            )
            for b in range(B)
        ]

    def issue_enc(c):
        return pltpu.async_copy(
            enc_hbm.at[pl.ds(e_base + c * CW, CW)], ebufs[c % 2], e_sems[c % 2]
        )

    in_descs = {0: issue_in(0), 1: issue_in(1)}
    enc_descs = {0: issue_enc(0)}
    out_descs = {}

    for c in range(N_CHUNKS):
        slot = c % NSLOT
        if c + 1 < N_CHUNKS:
            enc_descs[c + 1] = issue_enc(c + 1)
        for d in in_descs.pop(c):
            d.wait()
        enc_descs.pop(c).wait()

        tbuf = slots[slot]
        ebuf = ebufs[c % 2]

        def body(i, carry):
            base = i * (LANES * GROUP)
            for g in range(GROUP):
                off = base + g * LANES
                e = ebuf[pl.ds(off, LANES)]
                for b in range(B):
                    bo = b * CW + off
                    tbuf[pl.ds(bo, LANES)] = tbuf[pl.ds(bo, LANES)] + e
            return carry

        lax.fori_loop(0, CW // (LANES * GROUP), body, 0)

        if c + 2 < N_CHUNKS:
            if c - 1 >= 0:
                for d in out_descs[c - 1]:
                    d.wait()
            in_descs[c + 2] = issue_in(c + 2)

        e0 = e_base + c * CW
        out_descs[c] = [
            pltpu.async_copy(
                tbuf.at[pl.ds(b * CW, CW)],
                out_hbm.at[pl.ds(b * (S * D) + e0, CW)],
                out_sems[slot],
            )
            for b in range(B)
        ]

    for c in range(N_CHUNKS - 3, N_CHUNKS):
        for d in out_descs[c]:
            d.wait()


def kernel(token_embeddings, encoding):
    out = _pe_add(token_embeddings.reshape(-1), encoding.reshape(-1))
    return out.reshape(B, S, D)


# speedup vs baseline: 3.4457x; 2.8952x over previous
import functools

import jax
import jax.numpy as jnp
from jax import lax
from jax.experimental import pallas as pl
from jax.experimental.pallas import tpu as pltpu
from jax.experimental.pallas import tpu_sc as plsc

B, S, D = 4, 4096, 1024
NC, NS = 2, 16
NW = NC * NS
SEQ_PER_W = S // NW
CHUNK = 8
N_CHUNKS = SEQ_PER_W // CHUNK
SLICES = CHUNK * D // 16
LANES = 16
GROUP = 8
NSLOT = 3

_mesh = plsc.VectorSubcoreMesh(core_axis_name="c", subcore_axis_name="s")


@functools.partial(
    pl.kernel,
    out_type=jax.ShapeDtypeStruct((B, S, D), jnp.float32),
    mesh=_mesh,
    scratch_types=[
        [pltpu.VMEM((B * CHUNK, D), jnp.float32) for _ in range(NSLOT)],
        [pltpu.VMEM((CHUNK, D), jnp.float32) for _ in range(2)],
        [pltpu.SemaphoreType.DMA for _ in range(NSLOT)],
        [pltpu.SemaphoreType.DMA for _ in range(NSLOT)],
        [pltpu.SemaphoreType.DMA for _ in range(2)],
    ],
)
def _pe_add(te_hbm, enc_hbm, out_hbm, slots, ebufs, in_sems, out_sems, e_sems):
    wid = lax.axis_index("s") * NC + lax.axis_index("c")
    s_base = wid * SEQ_PER_W

    def issue_in(c):
        slot = c % NSLOT
        s0 = s_base + c * CHUNK
        return [
            pltpu.async_copy(
                te_hbm.at[b, pl.ds(s0, CHUNK)],
                slots[slot].at[pl.ds(b * CHUNK, CHUNK)],
                in_sems[slot],
            )
            for b in range(B)
        ]

    def issue_enc(c):
        return pltpu.async_copy(
            enc_hbm.at[pl.ds(s_base + c * CHUNK, CHUNK)], ebufs[c % 2], e_sems[c % 2]
        )

    in_descs = {0: issue_in(0), 1: issue_in(1)}
    enc_descs = {0: issue_enc(0)}
    out_descs = {}

    for c in range(N_CHUNKS):
        slot = c % NSLOT
        if c + 1 < N_CHUNKS:
            enc_descs[c + 1] = issue_enc(c + 1)
        for d in in_descs.pop(c):
            d.wait()
        enc_descs.pop(c).wait()

        tbuf = slots[slot]
        ebuf = ebufs[c % 2]

        @plsc.parallel_loop(0, SLICES, step=1, unroll=GROUP)
        def body(i):
            row = i >> 6
            col = (i & 63) * LANES
            e = ebuf[row, pl.ds(col, LANES)]
            for b in range(B):
                r = b * CHUNK + row
                tbuf[r, pl.ds(col, LANES)] = tbuf[r, pl.ds(col, LANES)] + e

        if c + 2 < N_CHUNKS:
            if c - 1 >= 0:
                for d in out_descs[c - 1]:
                    d.wait()
            in_descs[c + 2] = issue_in(c + 2)

        s0 = s_base + c * CHUNK
        out_descs[c] = [
            pltpu.async_copy(
                tbuf.at[pl.ds(b * CHUNK, CHUNK)],
                out_hbm.at[b, pl.ds(s0, CHUNK)],
                out_sems[slot],
            )
            for b in range(B)
        ]

    for c in range(N_CHUNKS - 3, N_CHUNKS):
        for d in out_descs[c]:
            d.wait()


def kernel(token_embeddings, encoding):
    return _pe_add(token_embeddings, encoding)


# speedup vs baseline: 5.5223x; 1.6027x over previous
import jax
import jax.numpy as jnp
from jax.experimental import pallas as pl

B, S, D = 4, 4096, 1024
BS = 256


def _body(te_ref, enc_ref, out_ref):
    out_ref[...] = te_ref[...] + enc_ref[...][None]


def kernel(token_embeddings, encoding):
    return pl.pallas_call(
        _body,
        grid=(S // BS,),
        in_specs=[
            pl.BlockSpec((B, BS, D), lambda i: (0, i, 0)),
            pl.BlockSpec((BS, D), lambda i: (i, 0)),
        ],
        out_specs=pl.BlockSpec((B, BS, D), lambda i: (0, i, 0)),
        out_shape=jax.ShapeDtypeStruct((B, S, D), jnp.float32),
    )(token_embeddings, encoding)
